# initial kernel scaffold (unmeasured)
import jax
import jax.numpy as jnp
from jax import lax
from jax.experimental import pallas as pl
from jax.experimental.pallas import tpu as pltpu

N_DEV = 4


def kernel(x, w_mat):
    m_total, k_shard = x.shape
    k_total, n_out = w_mat.shape
    m_blk = m_total // N_DEV

    def body(x_ref, w_ref, out_ref, recv_ref, amax_ref,
             send_sems, recv_sems, amax_send_sems, amax_recv_sems):
        my_i = lax.axis_index("i")

        barrier_sem = pltpu.get_barrier_semaphore()
        for h in range(1, N_DEV):
            peer = lax.rem(my_i + h, N_DEV)
            pl.semaphore_signal(
                barrier_sem, inc=1,
                device_id=(peer,), device_id_type=pl.DeviceIdType.MESH,
            )
        pl.semaphore_wait(barrier_sem, N_DEV - 1)

        rdmas = []
        for h in range(1, N_DEV):
            peer = lax.rem(my_i + h, N_DEV)
            rdma = pltpu.make_async_remote_copy(
                src_ref=x_ref.at[pl.ds(peer * m_blk, m_blk), :],
                dst_ref=recv_ref.at[h - 1],
                send_sem=send_sems.at[h - 1],
                recv_sem=recv_sems.at[h - 1],
                device_id=(peer,),
                device_id_type=pl.DeviceIdType.MESH,
            )
            rdma.start()
            rdmas.append(rdma)

        x_loc = x_ref[pl.ds(my_i * m_blk, m_blk), :]
        w_loc = w_ref[pl.ds(my_i * k_shard, k_shard), :]
        out_ref[...] = jnp.dot(x_loc, w_loc, preferred_element_type=jnp.float32)

        for h in range(1, N_DEV):
            src = lax.rem(my_i - h + N_DEV, N_DEV)
            rdmas[h - 1].wait_recv()
            w_blk = w_ref[pl.ds(src * k_shard, k_shard), :]
            out_ref[...] += jnp.dot(
                recv_ref[h - 1], w_blk, preferred_element_type=jnp.float32
            )
        for r in rdmas:
            r.wait_send()

        y = jnp.maximum(out_ref[...], 0.0)
        out_ref[...] = y
        amax_ref[N_DEV - 1, :] = jnp.full((128,), jnp.max(y), jnp.float32)

        amax_rdmas = []
        for h in range(1, N_DEV):
            peer = lax.rem(my_i + h, N_DEV)
            r = pltpu.make_async_remote_copy(
                src_ref=amax_ref.at[N_DEV - 1],
                dst_ref=amax_ref.at[h - 1],
                send_sem=amax_send_sems.at[h - 1],
                recv_sem=amax_recv_sems.at[h - 1],
                device_id=(peer,),
                device_id_type=pl.DeviceIdType.MESH,
            )
            r.start()
            amax_rdmas.append(r)
        for r in amax_rdmas:
            r.wait()

        scale = jnp.max(amax_ref[...]) / 448.0
        q = (out_ref[...] / scale).astype(jnp.float8_e4m3fn)
        out_ref[...] = q.astype(jnp.float32) * scale

    return pl.pallas_call(
        body,
        out_shape=jax.ShapeDtypeStruct((m_blk, n_out), jnp.float32),
        in_specs=[
            pl.BlockSpec(memory_space=pltpu.VMEM),
            pl.BlockSpec(memory_space=pltpu.VMEM),
        ],
        out_specs=pl.BlockSpec(memory_space=pltpu.VMEM),
        scratch_shapes=[
            pltpu.VMEM((N_DEV - 1, m_blk, k_shard), jnp.float32),
            pltpu.VMEM((N_DEV, 128), jnp.float32),
            pltpu.SemaphoreType.DMA((N_DEV - 1,)),
            pltpu.SemaphoreType.DMA((N_DEV - 1,)),
            pltpu.SemaphoreType.DMA((N_DEV - 1,)),
            pltpu.SemaphoreType.DMA((N_DEV - 1,)),
        ],
        compiler_params=pltpu.CompilerParams(collective_id=0),
    )(x, w_mat)


# baseline (device time: 122784 ns/iter reference)
import jax
import jax.numpy as jnp
from jax import lax
from jax.experimental import pallas as pl
from jax.experimental.pallas import tpu as pltpu

N_DEV = 4


def kernel(x, w_mat):
    m_total, k_shard = x.shape
    k_total, n_out = w_mat.shape
    m_blk = m_total // N_DEV

    def body(x_ref, w_ref, out_ref, recv_ref, x_loc_ref, w_buf_ref, amax_ref,
             send_sems, recv_sems, amax_send_sems, amax_recv_sems,
             loc_sem, w_sems):
        my_i = lax.axis_index("i")

        barrier_sem = pltpu.get_barrier_semaphore()
        for h in range(1, N_DEV):
            peer = lax.rem(my_i + h, N_DEV)
            pl.semaphore_signal(
                barrier_sem, inc=1,
                device_id=(peer,), device_id_type=pl.DeviceIdType.MESH,
            )
        pl.semaphore_wait(barrier_sem, N_DEV - 1)

        rdmas = []
        for h in range(1, N_DEV):
            peer = lax.rem(my_i + h, N_DEV)
            rdma = pltpu.make_async_remote_copy(
                src_ref=x_ref.at[pl.ds(peer * m_blk, m_blk), :],
                dst_ref=recv_ref.at[h - 1],
                send_sem=send_sems.at[h - 1],
                recv_sem=recv_sems.at[h - 1],
                device_id=(peer,),
                device_id_type=pl.DeviceIdType.MESH,
            )
            rdma.start()
            rdmas.append(rdma)

        def w_src(step):
            j = lax.rem(my_i - step + N_DEV, N_DEV)
            return w_ref.at[pl.ds(j * k_shard, k_shard), :]

        loc_cp = pltpu.make_async_copy(
            x_ref.at[pl.ds(my_i * m_blk, m_blk), :], x_loc_ref, loc_sem
        )
        loc_cp.start()
        w_cps = {}
        for step in range(2):
            w_cps[step] = pltpu.make_async_copy(
                w_src(step), w_buf_ref.at[step % 2], w_sems.at[step % 2]
            )
            w_cps[step].start()

        loc_cp.wait()
        w_cps[0].wait()
        out_ref[...] = jnp.dot(
            x_loc_ref[...], w_buf_ref[0], preferred_element_type=jnp.float32
        )

        for h in range(1, N_DEV):
            rdmas[h - 1].wait_recv()
            w_cps[h].wait()
            if h + 1 < N_DEV:
                w_cps[h + 1] = pltpu.make_async_copy(
                    w_src(h + 1), w_buf_ref.at[(h + 1) % 2], w_sems.at[(h + 1) % 2]
                )
                w_cps[h + 1].start()
            out_ref[...] += jnp.dot(
                recv_ref[h - 1], w_buf_ref[h % 2],
                preferred_element_type=jnp.float32,
            )
        for r in rdmas:
            r.wait_send()

        out_ref[...] = jnp.maximum(out_ref[...], 0.0)
        amax_ref[N_DEV - 1, :] = jnp.full((128,), jnp.max(out_ref[...]),
                                          jnp.float32)

        amax_rdmas = []
        for h in range(1, N_DEV):
            peer = lax.rem(my_i + h, N_DEV)
            r = pltpu.make_async_remote_copy(
                src_ref=amax_ref.at[N_DEV - 1],
                dst_ref=amax_ref.at[h - 1],
                send_sem=amax_send_sems.at[h - 1],
                recv_sem=amax_recv_sems.at[h - 1],
                device_id=(peer,),
                device_id_type=pl.DeviceIdType.MESH,
            )
            r.start()
            amax_rdmas.append(r)
        for r in amax_rdmas:
            r.wait()

        scale = jnp.max(amax_ref[...]) / 448.0
        q = (out_ref[...] / scale).astype(jnp.float8_e4m3fn)
        out_ref[...] = q.astype(jnp.float32) * scale

    return pl.pallas_call(
        body,
        out_shape=jax.ShapeDtypeStruct((m_blk, n_out), jnp.float32),
        in_specs=[
            pl.BlockSpec(memory_space=pl.ANY),
            pl.BlockSpec(memory_space=pl.ANY),
        ],
        out_specs=pl.BlockSpec(memory_space=pltpu.VMEM),
        scratch_shapes=[
            pltpu.VMEM((N_DEV - 1, m_blk, k_shard), jnp.float32),
            pltpu.VMEM((m_blk, k_shard), jnp.float32),
            pltpu.VMEM((2, k_shard, n_out), jnp.float32),
            pltpu.VMEM((N_DEV, 128), jnp.float32),
            pltpu.SemaphoreType.DMA((N_DEV - 1,)),
            pltpu.SemaphoreType.DMA((N_DEV - 1,)),
            pltpu.SemaphoreType.DMA((N_DEV - 1,)),
            pltpu.SemaphoreType.DMA((N_DEV - 1,)),
            pltpu.SemaphoreType.DMA(()),
            pltpu.SemaphoreType.DMA((2,)),
        ],
        compiler_params=pltpu.CompilerParams(
            collective_id=0, vmem_limit_bytes=100 * 1024 * 1024
        ),
    )(x, w_mat)


# device time: 80952 ns/iter; 1.5168x vs baseline; 1.5168x over previous
import jax
import jax.numpy as jnp
from jax import lax
from jax.experimental import pallas as pl
from jax.experimental.pallas import tpu as pltpu

N_DEV = 4
CAST_ORDER = (1, 3, 0, 2)
STEP_OFFS = (0, 3, 1, 2)


def kernel(x, w_mat):
    m_total, k_shard = x.shape
    k_total, n_out = w_mat.shape
    m_blk = m_total // N_DEV

    def body(x_ref, w_ref, out_ref, xs_ref, xb_ref, recv_ref, w_buf_ref,
             amax_ref, send_sems, recv_sems, amax_send_sems, amax_recv_sems,
             xs_sems, w_sems):
        my_i = lax.axis_index("i")

        barrier_sem = pltpu.get_barrier_semaphore()
        for h in range(1, N_DEV):
            peer = lax.rem(my_i + h, N_DEV)
            pl.semaphore_signal(
                barrier_sem, inc=1,
                device_id=(peer,), device_id_type=pl.DeviceIdType.MESH,
            )
        pl.semaphore_wait(barrier_sem, N_DEV - 1)

        def w_cp(step):
            j = lax.rem(my_i + STEP_OFFS[step], N_DEV)
            return pltpu.make_async_copy(
                w_ref.at[pl.ds(j * k_shard, k_shard), :],
                w_buf_ref.at[step % 2],
                w_sems.at[step % 2],
            )

        w_cps = [w_cp(0), w_cp(1)]
        w_cps[0].start()
        w_cps[1].start()

        def x_cp(idx, d):
            return pltpu.make_async_copy(
                x_ref.at[pl.ds(lax.rem(my_i + d, N_DEV) * m_blk, m_blk), :],
                xs_ref.at[idx % 2],
                xs_sems.at[idx % 2],
            )

        x_cps = {}
        for idx in range(2):
            d = CAST_ORDER[idx]
            x_cps[d] = x_cp(idx, d)
            x_cps[d].start()

        def remote(d):
            return pltpu.make_async_remote_copy(
                src_ref=xb_ref.at[d],
                dst_ref=recv_ref.at[d - 1],
                send_sem=send_sems.at[d - 1],
                recv_sem=recv_sems.at[d - 1],
                device_id=(lax.rem(my_i + d, N_DEV),),
                device_id_type=pl.DeviceIdType.MESH,
            )

        rdmas = {}
        for idx, d in enumerate(CAST_ORDER):
            x_cps[d].wait()
            xb_ref[d] = xs_ref[idx % 2].astype(jnp.bfloat16)
            if idx + 2 < N_DEV:
                dn = CAST_ORDER[idx + 2]
                x_cps[dn] = x_cp(idx + 2, dn)
                x_cps[dn].start()
            rdmas[d] = remote(d) if d else None
            if d in (1, 3):
                rdmas[d].start()

        w_cps[0].wait()
        out_ref[...] = jnp.dot(
            xb_ref[0].astype(jnp.float32), w_buf_ref[0],
            preferred_element_type=jnp.float32,
        )
        w_cps.append(w_cp(2))
        w_cps[2].start()

        rdmas[1].wait_send()
        rdmas[3].wait_send()
        rdmas[2].start()

        rdmas[1].wait_recv()
        w_cps[1].wait()
        out_ref[...] += jnp.dot(
            recv_ref[0].astype(jnp.float32), w_buf_ref[1],
            preferred_element_type=jnp.float32,
        )
        w_cps.append(w_cp(3))
        w_cps[3].start()

        rdmas[3].wait_recv()
        w_cps[2].wait()
        out_ref[...] += jnp.dot(
            recv_ref[2].astype(jnp.float32), w_buf_ref[0],
            preferred_element_type=jnp.float32,
        )

        rdmas[2].wait_recv()
        w_cps[3].wait()
        out_ref[...] += jnp.dot(
            recv_ref[1].astype(jnp.float32), w_buf_ref[1],
            preferred_element_type=jnp.float32,
        )
        rdmas[2].wait_send()

        out_ref[...] = jnp.maximum(out_ref[...], 0.0)
        amax_ref[N_DEV - 1, :] = jnp.full((128,), jnp.max(out_ref[...]),
                                          jnp.float32)

        amax_rdmas = []
        for h in range(1, N_DEV):
            peer = lax.rem(my_i + h, N_DEV)
            r = pltpu.make_async_remote_copy(
                src_ref=amax_ref.at[N_DEV - 1],
                dst_ref=amax_ref.at[h - 1],
                send_sem=amax_send_sems.at[h - 1],
                recv_sem=amax_recv_sems.at[h - 1],
                device_id=(peer,),
                device_id_type=pl.DeviceIdType.MESH,
            )
            r.start()
            amax_rdmas.append(r)
        for r in amax_rdmas:
            r.wait()

        scale = jnp.max(amax_ref[...]) / 448.0
        q = (out_ref[...] / scale).astype(jnp.float8_e4m3fn)
        out_ref[...] = q.astype(jnp.float32) * scale

    return pl.pallas_call(
        body,
        out_shape=jax.ShapeDtypeStruct((m_blk, n_out), jnp.float32),
        in_specs=[
            pl.BlockSpec(memory_space=pl.ANY),
            pl.BlockSpec(memory_space=pl.ANY),
        ],
        out_specs=pl.BlockSpec(memory_space=pltpu.VMEM),
        scratch_shapes=[
            pltpu.VMEM((2, m_blk, k_shard), jnp.float32),
            pltpu.VMEM((N_DEV, m_blk, k_shard), jnp.bfloat16),
            pltpu.VMEM((N_DEV - 1, m_blk, k_shard), jnp.bfloat16),
            pltpu.VMEM((2, k_shard, n_out), jnp.float32),
            pltpu.VMEM((N_DEV, 128), jnp.float32),
            pltpu.SemaphoreType.DMA((N_DEV - 1,)),
            pltpu.SemaphoreType.DMA((N_DEV - 1,)),
            pltpu.SemaphoreType.DMA((N_DEV - 1,)),
            pltpu.SemaphoreType.DMA((N_DEV - 1,)),
            pltpu.SemaphoreType.DMA((2,)),
            pltpu.SemaphoreType.DMA((2,)),
        ],
        compiler_params=pltpu.CompilerParams(
            collective_id=0, vmem_limit_bytes=100 * 1024 * 1024
        ),
    )(x, w_mat)
